# baseline (speedup 1.0000x reference)
import jax
import jax.numpy as jnp
from jax import lax
from jax.experimental import pallas as pl
from jax.experimental.pallas import tpu as pltpu
from jax.experimental.pallas import tpu_sc as plsc

B = 32
D = 1024
V = 100000
TOP_P = 0.9

CH = 6400
VROW = 16 * CH
BN = 4096
WIN = 640
NWIN = CH // WIN
GRP = 8
NST = WIN * 16 // 128
DW = 4096
NDW = VROW // DW
PASSES = ((0, 2048, 0x7FF), (11, 2048, 0x7FF), (22, 1024, 0x3FF))

_MESH = plsc.VectorSubcoreMesh(core_axis_name="c", subcore_axis_name="s")
_CP = pltpu.CompilerParams(needs_layout_passes=False)


def _matmul_body(x_ref, w_ref, b_ref, o_ref):
    o_ref[...] = (
        lax.dot_general(
            x_ref[...], w_ref[...], (((1,), (0,)), ((), ())),
            preferred_element_type=jnp.float32,
        )
        + b_ref[...]
    )


def _logits(x, W, b):
    grid = VROW // BN
    return pl.pallas_call(
        _matmul_body,
        grid=(grid,),
        in_specs=[
            pl.BlockSpec((B, D), lambda i: (0, 0)),
            pl.BlockSpec((D, BN), lambda i: (0, i)),
            pl.BlockSpec((BN,), lambda i: (i,)),
        ],
        out_specs=pl.BlockSpec((B, BN), lambda i: (0, i)),
        out_shape=jax.ShapeDtypeStruct((B, VROW), jnp.float32),
    )(x, W, b)


def _keys_body(l_ref, t_ref):
    x = l_ref[...]
    valid = lax.broadcasted_iota(jnp.int32, (B, VROW), 1) < V
    x = jnp.where(valid, x, -jnp.inf)
    xmax = jnp.max(x, axis=1, keepdims=True)
    u = jnp.exp(x - xmax)
    s = jnp.sum(u, axis=1, keepdims=True)
    p = u / s
    t_ref[...] = ~lax.bitcast_convert_type(p, jnp.int32)


def _keys(logits):
    return pl.pallas_call(
        _keys_body,
        out_shape=jax.ShapeDtypeStruct((B, VROW), jnp.int32),
    )(logits)


def _worker_id():
    return lax.axis_index("s") * 2 + lax.axis_index("c")


def _load_regions(src, dst, w, rowbase, sem):
    cps = [
        pltpu.make_async_copy(
            src.at[pl.ds(rowbase + l * CH + w * WIN, WIN)],
            dst.at[pl.ds(l * WIN, WIN)], sem)
        for l in range(16)
    ]
    for c in cps:
        c.start()
    for c in cps:
        c.wait()


def _radix_pass_body(shift, nb, dmask, first,
                     src_t, src_i, dst_t, dst_i,
                     hist, base2, tbuf, ibuf, posst, tst, ist,
                     sem_ld, sem_st):
    rowbase = _worker_id() * VROW
    iota = lax.iota(jnp.int32, 16)
    ones = jnp.ones((16,), jnp.int32)
    zeros = jnp.zeros((16,), jnp.int32)

    def zb(i, _):
        hist[pl.ds(i * 16, 16)] = zeros
        return 0
    lax.fori_loop(0, nb, zb, 0)

    def aw(w, _):
        _load_regions(src_t, tbuf, w, rowbase, sem_ld)

        def ha(i, _):
            for dj in range(GRP):
                j = i * GRP + dj
                tv = plsc.load_gather(tbuf, [iota * WIN + j])
                d = lax.shift_right_logical(tv, shift) & dmask
                plsc.addupdate_scatter(hist, [iota * nb + d], ones)
            return 0
        lax.fori_loop(0, WIN // GRP, ha, 0)
        return 0
    lax.fori_loop(0, NWIN, aw, 0)

    def b1(g, tot):
        dg = g * 16 + iota
        acc = plsc.load_gather(hist, [dg])
        for l in range(1, 16):
            acc = acc + plsc.load_gather(hist, [l * nb + dg])
        incl = plsc.cumsum(acc)
        base2[pl.ds(g * 16, 16)] = (incl - acc) + tot
        return tot + (zeros + jnp.sum(acc))
    lax.fori_loop(0, nb // 16, b1, zeros)

    def b2(g, _):
        dg = g * 16 + iota
        acc = base2[pl.ds(g * 16, 16)]
        for l in range(16):
            cnt = plsc.load_gather(hist, [l * nb + dg])
            plsc.store_scatter(hist, [l * nb + dg], acc)
            acc = acc + cnt
        return 0
    lax.fori_loop(0, nb // 16, b2, 0)

    def cw(w, _):
        _load_regions(src_t, tbuf, w, rowbase, sem_ld)
        if not first:
            _load_regions(src_i, ibuf, w, rowbase, sem_ld)

        def hc(i, _):
            for dj in range(GRP):
                j = i * GRP + dj
                tv = plsc.load_gather(tbuf, [iota * WIN + j])
                if first:
                    iv = iota * CH + (w * WIN + j)
                else:
                    iv = plsc.load_gather(ibuf, [iota * WIN + j])
                d = lax.shift_right_logical(tv, shift) & dmask
                addr = iota * nb + d
                pos = plsc.load_gather(hist, [addr])
                plsc.store_scatter(hist, [addr], pos + ones)
                posst[i, pl.ds(dj * 16, 16)] = pos + rowbase
                tst[i, pl.ds(dj * 16, 16)] = tv
                ist[i, pl.ds(dj * 16, 16)] = iv
            return 0
        lax.fori_loop(0, WIN // GRP, hc, 0)

        def fire(c, _):
            pltpu.make_async_copy(
                tst.at[c], dst_t.at[posst.at[c]], sem_st).start()
            pltpu.make_async_copy(
                ist.at[c], dst_i.at[posst.at[c]], sem_st).start()
            return 0
        lax.fori_loop(0, NST, fire, 0)

        def drain(c, _):
            pltpu.make_async_copy(
                tst.at[c], dst_t.at[posst.at[c]], sem_st).wait()
            pltpu.make_async_copy(
                ist.at[c], dst_i.at[posst.at[c]], sem_st).wait()
            return 0
        lax.fori_loop(0, NST, drain, 0)
        return 0
    lax.fori_loop(0, NWIN, cw, 0)


_RADIX_SCRATCH = [
    pltpu.VMEM((32768,), jnp.int32),
    pltpu.VMEM((2048,), jnp.int32),
    pltpu.VMEM((16 * WIN,), jnp.int32),
    pltpu.VMEM((16 * WIN,), jnp.int32),
    pltpu.VMEM((NST, 128), jnp.int32),
    pltpu.VMEM((NST, 128), jnp.int32),
    pltpu.VMEM((NST, 128), jnp.int32),
    pltpu.SemaphoreType.DMA,
    pltpu.SemaphoreType.DMA,
]
_PAIR = [jax.ShapeDtypeStruct((B * VROW,), jnp.int32)] * 2


def _pass0(t_flat):
    sh, nb, mk = PASSES[0]

    def body(src_t, dst_t, dst_i, *scr):
        _radix_pass_body(sh, nb, mk, True, src_t, None, dst_t, dst_i, *scr)

    return pl.kernel(body, out_type=_PAIR, mesh=_MESH, compiler_params=_CP,
                     scratch_types=_RADIX_SCRATCH)(t_flat)


def _passk(k, src_t, src_i):
    sh, nb, mk = PASSES[k]

    def body(st, si, dst_t, dst_i, *scr):
        _radix_pass_body(sh, nb, mk, False, st, si, dst_t, dst_i, *scr)

    return pl.kernel(body, out_type=_PAIR, mesh=_MESH, compiler_params=_CP,
                     scratch_types=_RADIX_SCRATCH)(src_t, src_i)


def _sample_body(ts, g_in, jst_out, persd, dts, ddg, j16, sem_ld, sem_st):
    wid = _worker_id()
    rowbase = wid * VROW
    iota = lax.iota(jnp.int32, 16)
    zeros = jnp.zeros((16,), jnp.int32)
    fzeros = jnp.zeros((16,), jnp.float32)

    def d1(w, _):
        cp = pltpu.make_async_copy(
            ts.at[pl.ds(rowbase + w * DW, DW)], dts, sem_ld)
        cp.start()
        cp.wait()

        def inner(g, _):
            basei = (g * 16 + iota) * 16
            acc = plsc.bitcast(~plsc.load_gather(dts, [basei]), jnp.float32)
            for k in range(1, 16):
                acc = acc + plsc.bitcast(
                    ~plsc.load_gather(dts, [basei + k]), jnp.float32)
            persd[pl.ds((w * (DW // 256) + g) * 16, 16)] = acc
            return 0
        lax.fori_loop(0, DW // 256, inner, 0)
        return 0
    lax.fori_loop(0, NDW, d1, 0)

    def d2(g, tot):
        pv = persd[pl.ds(g * 16, 16)]
        incl = plsc.cumsum(pv)
        persd[pl.ds(g * 16, 16)] = (incl - pv) + tot
        return tot + (fzeros + jnp.sum(pv))
    lax.fori_loop(0, VROW // 256, d2, fzeros)

    cp0 = pltpu.make_async_copy(
        ts.at[pl.ds(rowbase, 16)], dts.at[pl.ds(0, 16)], sem_ld)
    cp0.start()
    cp0.wait()
    p0 = plsc.bitcast(~plsc.load_gather(dts, [zeros]), jnp.float32)
    peff = jnp.maximum(p0, jnp.float32(TOP_P))

    def d3(w, carry):
        vm, vidx = carry
        ct = pltpu.make_async_copy(
            ts.at[pl.ds(rowbase + w * DW, DW)], dts, sem_ld)
        cg = pltpu.make_async_copy(
            g_in.at[pl.ds(rowbase + w * DW, DW)], ddg, sem_ld)
        ct.start()
        cg.start()
        ct.wait()
        cg.wait()

        def inner(v, carry2):
            vm2, vidx2 = carry2
            pv = plsc.bitcast(~dts[pl.ds(v * 16, 16)], jnp.float32)
            pref = persd[pl.ds(w * (DW // 16) + v, 16)][0]
            cums = plsc.cumsum(pv) + pref
            gv = ddg[pl.ds(v * 16, 16)]
            val = jnp.where(cums <= peff, pv * jnp.exp(gv),
                            jnp.float32(-1.0))
            posv = iota + (w * DW + v * 16)
            upd = val > vm2
            return (jnp.where(upd, val, vm2), jnp.where(upd, posv, vidx2))
        return lax.fori_loop(0, DW // 16, inner, (vm, vidx))

    vm0 = jnp.full((16,), -3.0, jnp.float32)
    vi0 = jnp.zeros((16,), jnp.int32)
    vm, vidx = lax.fori_loop(0, NDW, d3, (vm0, vi0))
    m = jnp.max(vm)
    cand = jnp.where(vm == m, vidx, jnp.int32(2**30))
    jstar = jnp.min(cand)
    j16[...] = zeros + jstar
    co = pltpu.make_async_copy(j16, jst_out.at[pl.ds(wid * 16, 16)], sem_st)
    co.start()
    co.wait()


def _sample(ts, g_flat):
    return pl.kernel(
        _sample_body,
        out_type=[jax.ShapeDtypeStruct((B * 16,), jnp.int32)],
        mesh=_MESH,
        compiler_params=_CP,
        scratch_types=[
            pltpu.VMEM((VROW // 16 + 16,), jnp.float32),
            pltpu.VMEM((DW,), jnp.int32),
            pltpu.VMEM((DW,), jnp.float32),
            pltpu.VMEM((16,), jnp.int32),
            pltpu.SemaphoreType.DMA,
            pltpu.SemaphoreType.DMA,
        ],
    )(ts, g_flat)[0]


def kernel(batch, W, b):
    x = batch[:, -1, :]
    bpad = jnp.pad(b, (0, VROW - V))
    logits = _logits(x, W, bpad)
    t = _keys(logits)

    keys = jax.random.split(jax.random.key(42), B)
    g = jax.vmap(lambda k: jax.random.gumbel(k, (V,), jnp.float32))(keys)
    g = jnp.pad(g, ((0, 0), (0, VROW - V)))

    t0, i0 = _pass0(t.reshape(-1))
    t1, i1 = _passk(1, t0, i0)
    ts, order = _passk(2, t1, i1)
    jst = _sample(ts, g.reshape(-1))

    order = order.reshape(B, VROW)
    jstar = jst.reshape(B, 16)[:, 0]
    return jnp.take_along_axis(order, jstar[:, None], axis=1)[:, 0]
